# Initial kernel scaffold; baseline (speedup 1.0000x reference)
import jax
import jax.numpy as jnp
from jax.experimental import pallas as pl


def kernel(x, edge_index, edge_attr, edge_t, edge_d, emb_type, emb_dir, t_w, t_b, l0_wl, l0_bl, l0_wr, l0_br, g0, b0, l1_wl, l1_bl, l1_wr, l1_br, g1, b1):
    raise NotImplementedError("write your pallas kernel here")


# speedup vs baseline: 3.3011x; 3.3011x over previous
import functools

import jax
import jax.numpy as jnp
from jax import lax
from jax.experimental import pallas as pl
from jax.experimental.pallas import tpu as pltpu
from jax.experimental.pallas import tpu_sc as plsc

N = 10000
E = 160000
DIN = 256
DH = 512
DOUT = 128
DE = 50
DT = 50

NCORE = 2
NSUB = 16
K = 128
CHUNKS = 40
E_PAD = NCORE * NSUB * CHUNKS * K
N_PAD = 10240
ROWS_PER_SUB = N_PAD // NSUB
DUMP_ROW = N + 64
EB = 1024
RB = 1024
GRID_E = E_PAD // EB
GRID_N = N_PAD // RB


def _sc_body_factory(tasks):
    ntask = len(tasks)

    def body(*refs):
        tables = refs[:ntask]
        src4 = refs[ntask]
        dst4 = refs[ntask + 1]
        zblk = refs[ntask + 2]
        outs = refs[ntask + 3:ntask + 3 + ntask]
        acc, src_all, dst_all, rows, sem = refs[ntask + 3 + ntask:]

        c = lax.axis_index("c")
        s = lax.axis_index("s")
        r0 = s * ROWS_PER_SUB

        pltpu.sync_copy(src4.at[c, s], src_all)
        pltpu.sync_copy(dst4.at[c, s], dst_all)

        for t, mode in enumerate(tasks):
            tbl = tables[t]
            out = outs[t]
            pltpu.sync_copy(zblk, acc.at[pl.ds(r0, ROWS_PER_SUB)])
            plsc.subcore_barrier()

            def chunk(j, carry):
                if mode == "linear":
                    pltpu.sync_copy(tbl.at[c, s, j], rows)
                else:
                    pltpu.async_copy(tbl.at[src_all.at[j]], rows, sem).wait()
                pltpu.sync_copy(rows, acc.at[dst_all.at[j]], add=True)
                return carry

            lax.fori_loop(0, CHUNKS, chunk, 0)
            plsc.subcore_barrier()
            pltpu.sync_copy(acc.at[pl.ds(r0, ROWS_PER_SUB)],
                            out.at[c, pl.ds(r0, ROWS_PER_SUB)])
            plsc.subcore_barrier()

    return body


def _make_seg_sum(tasks):
    mesh = plsc.VectorSubcoreMesh(core_axis_name="c", subcore_axis_name="s")
    ntask = len(tasks)
    out_type = [jax.ShapeDtypeStruct((NCORE, N_PAD, 128), jnp.float32)
                for _ in range(ntask)]
    scratch = [
        pltpu.VMEM_SHARED((N_PAD, 128), jnp.float32),
        pltpu.VMEM((CHUNKS, K), jnp.int32),
        pltpu.VMEM((CHUNKS, K), jnp.int32),
        pltpu.VMEM((K, 128), jnp.float32),
        pltpu.SemaphoreType.DMA,
    ]
    return pl.kernel(
        _sc_body_factory(tasks),
        out_type=out_type,
        mesh=mesh,
        scratch_types=scratch,
    )


def _feat_body(t_ref, a_ref, d_ref, embt_ref, embd_ref, wrow_ref, brow_ref,
               metmask_ref, cnt_ref, f_ref):
    t = t_ref[...]
    a = a_ref[...]
    d = d_ref[...]
    ia = lax.broadcasted_iota(jnp.int32, (1, 16), 1)
    idd = lax.broadcasted_iota(jnp.int32, (1, 8), 1)
    oh_a = (a == ia).astype(jnp.float32)
    oh_d = (d == idd).astype(jnp.float32)
    ef = jnp.dot(oh_a, embt_ref[...], preferred_element_type=jnp.float32)
    ef = ef + jnp.dot(oh_d, embd_ref[...], preferred_element_type=jnp.float32)
    et = jnp.cos(t * wrow_ref[...] + brow_ref[...]) * metmask_ref[...]
    f_ref[...] = ef + et + cnt_ref[...]


def _dense0_body(x_ref, pf_ref, px0_ref, px1_ref, wh0_ref, wh1_ref, wf_ref,
                 wr_ref, bias_ref, out_ref, s_ref, ss_ref):
    i = pl.program_id(0)
    sf = pf_ref[0] + pf_ref[1]
    sx0 = px0_ref[0] + px0_ref[1]
    sx1 = px1_ref[0] + px1_ref[1]
    inv = 1.0 / jnp.maximum(sf[:, 127:128], 1.0)
    out = jnp.dot(sx0 * inv, wh0_ref[...], preferred_element_type=jnp.float32)
    out += jnp.dot(sx1 * inv, wh1_ref[...], preferred_element_type=jnp.float32)
    out += jnp.dot(sf * inv, wf_ref[...], preferred_element_type=jnp.float32)
    out += jnp.dot(x_ref[...], wr_ref[...], preferred_element_type=jnp.float32)
    out += bias_ref[...]
    out_ref[...] = out
    ridx = i * RB + lax.broadcasted_iota(jnp.int32, (RB, 1), 0)
    valid = (ridx < N).astype(jnp.float32)
    ov = out * valid

    @pl.when(i == 0)
    def _():
        s_ref[...] = jnp.zeros_like(s_ref)
        ss_ref[...] = jnp.zeros_like(ss_ref)

    s_ref[...] += jnp.sum(ov, axis=0, keepdims=True)
    ss_ref[...] += jnp.sum(ov * ov, axis=0, keepdims=True)


def _bn_elu_p_body(out_ref, s_ref, ss_ref, g_ref, b_ref, w1h_ref,
                   h_ref, p_ref):
    m = s_ref[...] / N
    v = ss_ref[...] / N - m * m
    xn = (out_ref[...] - m) * lax.rsqrt(v + 1e-5) * g_ref[...] + b_ref[...]
    h = jnp.where(xn > 0, xn, jnp.exp(xn) - 1.0)
    h_ref[...] = h
    p_ref[...] = jnp.dot(h, w1h_ref[...], preferred_element_type=jnp.float32)


def _dense1_body(h_ref, pf_ref, pp_ref, wf1_ref, wr1_ref, bias_ref,
                 out_ref, s_ref, ss_ref):
    i = pl.program_id(0)
    sf = pf_ref[0] + pf_ref[1]
    sp = pp_ref[0] + pp_ref[1]
    inv = 1.0 / jnp.maximum(sf[:, 127:128], 1.0)
    out = sp * inv
    out += jnp.dot(sf * inv, wf1_ref[...], preferred_element_type=jnp.float32)
    out += jnp.dot(h_ref[...], wr1_ref[...], preferred_element_type=jnp.float32)
    out += bias_ref[...]
    out_ref[...] = out
    ridx = i * RB + lax.broadcasted_iota(jnp.int32, (RB, 1), 0)
    valid = (ridx < N).astype(jnp.float32)
    ov = out * valid

    @pl.when(i == 0)
    def _():
        s_ref[...] = jnp.zeros_like(s_ref)
        ss_ref[...] = jnp.zeros_like(ss_ref)

    s_ref[...] += jnp.sum(ov, axis=0, keepdims=True)
    ss_ref[...] += jnp.sum(ov * ov, axis=0, keepdims=True)


def _bn_elu_lsm_body(out_ref, s_ref, ss_ref, g_ref, b_ref, y_ref):
    m = s_ref[...] / N
    v = ss_ref[...] / N - m * m
    xn = (out_ref[...] - m) * lax.rsqrt(v + 1e-5) * g_ref[...] + b_ref[...]
    h = jnp.where(xn > 0, xn, jnp.exp(xn) - 1.0)
    mx = jnp.max(h, axis=1, keepdims=True)
    z = h - mx
    lse = jnp.log(jnp.sum(jnp.exp(z), axis=1, keepdims=True))
    y_ref[...] = z - lse


def kernel(x, edge_index, edge_attr, edge_t, edge_d, emb_type, emb_dir, t_w,
           t_b, l0_wl, l0_bl, l0_wr, l0_br, g0, b0, l1_wl, l1_bl, l1_wr,
           l1_br, g1, b1):
    f32 = jnp.float32

    src = edge_index[0].astype(jnp.int32)
    dst = edge_index[1].astype(jnp.int32)
    npad_e = E_PAD - E
    src4 = jnp.concatenate([src, jnp.zeros((npad_e,), jnp.int32)]) \
        .reshape(NCORE, NSUB, CHUNKS, K)
    dst4 = jnp.concatenate([dst, jnp.full((npad_e,), DUMP_ROW, jnp.int32)]) \
        .reshape(NCORE, NSUB, CHUNKS, K)
    t_pad = jnp.concatenate([edge_t.astype(f32), jnp.zeros((npad_e,), f32)]) \
        .reshape(E_PAD, 1)
    a_pad = jnp.concatenate([edge_attr.astype(jnp.int32),
                             jnp.zeros((npad_e,), jnp.int32)]).reshape(E_PAD, 1)
    d_pad = jnp.concatenate([edge_d.astype(jnp.int32),
                             jnp.zeros((npad_e,), jnp.int32)]).reshape(E_PAD, 1)

    xp = jnp.concatenate([x.astype(f32), jnp.zeros((N_PAD - N, DIN), f32)])
    x0 = xp[:, :128]
    x1 = xp[:, 128:]

    embt = jnp.zeros((16, 128), f32).at[:12, :DE].set(emb_type.astype(f32))
    embd = jnp.zeros((8, 128), f32).at[:2, :DE].set(emb_dir.astype(f32))
    wrow = jnp.zeros((1, 128), f32).at[0, DE:DE + DT].set(t_w.astype(f32))
    brow = jnp.zeros((1, 128), f32).at[0, DE:DE + DT].set(t_b.astype(f32))
    metmask = jnp.zeros((1, 128), f32).at[0, DE:DE + DT].set(1.0)
    cntrow = jnp.zeros((1, 128), f32).at[0, 127].set(1.0)

    wh0 = l0_wl[:, 0:128].T.astype(f32)
    wh1 = l0_wl[:, 128:256].T.astype(f32)
    wf0 = jnp.zeros((128, DH), f32) \
        .at[:DE].set(l0_wl[:, DIN:DIN + DE].T) \
        .at[DE:DE + DT].set(l0_wl[:, DIN + DE:DIN + DE + DT].T)
    wr0 = l0_wr.T.astype(f32)
    bias0 = (l0_bl + l0_br).reshape(1, DH).astype(f32)
    w1h = l1_wl[:, :DH].T.astype(f32)
    wf1 = jnp.zeros((128, DOUT), f32) \
        .at[:DE].set(l1_wl[:, DH:DH + DE].T) \
        .at[DE:DE + DT].set(l1_wl[:, DH + DE:DH + DE + DT].T)
    wr1 = l1_wr.T.astype(f32)
    bias1 = (l1_bl + l1_br).reshape(1, DOUT).astype(f32)
    g0r = g0.reshape(1, DH).astype(f32)
    b0r = b0.reshape(1, DH).astype(f32)
    g1r = g1.reshape(1, DOUT).astype(f32)
    b1r = b1.reshape(1, DOUT).astype(f32)

    zblk = jnp.zeros((ROWS_PER_SUB, 128), f32)

    full = lambda shape: pl.BlockSpec(shape, lambda i: (0,) * len(shape))
    feat = pl.pallas_call(
        _feat_body,
        grid=(GRID_E,),
        in_specs=[
            pl.BlockSpec((EB, 1), lambda i: (i, 0)),
            pl.BlockSpec((EB, 1), lambda i: (i, 0)),
            pl.BlockSpec((EB, 1), lambda i: (i, 0)),
            full((16, 128)), full((8, 128)), full((1, 128)),
            full((1, 128)), full((1, 128)), full((1, 128)),
        ],
        out_specs=pl.BlockSpec((EB, 128), lambda i: (i, 0)),
        out_shape=jax.ShapeDtypeStruct((E_PAD, 128), f32),
    )
    F = feat(t_pad, a_pad, d_pad, embt, embd, wrow, brow, metmask, cntrow)
    F5 = F.reshape(NCORE, NSUB, CHUNKS, K, 128)

    seg3 = _make_seg_sum(("linear", "gather", "gather"))
    pF, pX0, pX1 = seg3(F5, x0, x1, src4, dst4, zblk)

    dense0 = pl.pallas_call(
        _dense0_body,
        grid=(GRID_N,),
        in_specs=[
            pl.BlockSpec((RB, DIN), lambda i: (i, 0)),
            pl.BlockSpec((NCORE, RB, 128), lambda i: (0, i, 0)),
            pl.BlockSpec((NCORE, RB, 128), lambda i: (0, i, 0)),
            pl.BlockSpec((NCORE, RB, 128), lambda i: (0, i, 0)),
            full((128, DH)), full((128, DH)), full((128, DH)),
            full((DIN, DH)), full((1, DH)),
        ],
        out_specs=[
            pl.BlockSpec((RB, DH), lambda i: (i, 0)),
            pl.BlockSpec((1, DH), lambda i: (0, 0)),
            pl.BlockSpec((1, DH), lambda i: (0, 0)),
        ],
        out_shape=[
            jax.ShapeDtypeStruct((N_PAD, DH), f32),
            jax.ShapeDtypeStruct((1, DH), f32),
            jax.ShapeDtypeStruct((1, DH), f32),
        ],
    )
    out0, s0, ss0 = dense0(xp, pF, pX0, pX1, wh0, wh1, wf0, wr0, bias0)

    bn0 = pl.pallas_call(
        _bn_elu_p_body,
        grid=(GRID_N,),
        in_specs=[
            pl.BlockSpec((RB, DH), lambda i: (i, 0)),
            full((1, DH)), full((1, DH)), full((1, DH)), full((1, DH)),
            full((DH, DOUT)),
        ],
        out_specs=[
            pl.BlockSpec((RB, DH), lambda i: (i, 0)),
            pl.BlockSpec((RB, DOUT), lambda i: (i, 0)),
        ],
        out_shape=[
            jax.ShapeDtypeStruct((N_PAD, DH), f32),
            jax.ShapeDtypeStruct((N_PAD, DOUT), f32),
        ],
    )
    h, p = bn0(out0, s0, ss0, g0r, b0r, w1h)

    seg1 = _make_seg_sum(("gather",))
    (pP,) = seg1(p, src4, dst4, zblk)

    dense1 = pl.pallas_call(
        _dense1_body,
        grid=(GRID_N,),
        in_specs=[
            pl.BlockSpec((RB, DH), lambda i: (i, 0)),
            pl.BlockSpec((NCORE, RB, 128), lambda i: (0, i, 0)),
            pl.BlockSpec((NCORE, RB, 128), lambda i: (0, i, 0)),
            full((128, DOUT)), full((DH, DOUT)), full((1, DOUT)),
        ],
        out_specs=[
            pl.BlockSpec((RB, DOUT), lambda i: (i, 0)),
            pl.BlockSpec((1, DOUT), lambda i: (0, 0)),
            pl.BlockSpec((1, DOUT), lambda i: (0, 0)),
        ],
        out_shape=[
            jax.ShapeDtypeStruct((N_PAD, DOUT), f32),
            jax.ShapeDtypeStruct((1, DOUT), f32),
            jax.ShapeDtypeStruct((1, DOUT), f32),
        ],
    )
    out1, s1, ss1 = dense1(h, pF, pP, wf1, wr1, bias1)

    bn1 = pl.pallas_call(
        _bn_elu_lsm_body,
        grid=(GRID_N,),
        in_specs=[
            pl.BlockSpec((RB, DOUT), lambda i: (i, 0)),
            full((1, DOUT)), full((1, DOUT)), full((1, DOUT)), full((1, DOUT)),
        ],
        out_specs=pl.BlockSpec((RB, DOUT), lambda i: (i, 0)),
        out_shape=jax.ShapeDtypeStruct((N_PAD, DOUT), f32),
    )
    y = bn1(out1, s1, ss1, g1r, b1r)
    return y[:N]


# speedup vs baseline: 3.4544x; 1.0465x over previous
import functools

import jax
import jax.numpy as jnp
from jax import lax
from jax.experimental import pallas as pl
from jax.experimental.pallas import tpu as pltpu
from jax.experimental.pallas import tpu_sc as plsc

N = 10000
E = 160000
DIN = 256
DH = 512
DOUT = 128
DE = 50
DT = 50

NCORE = 2
NSUB = 16
K = 128
CHUNKS = 40
E_PAD = NCORE * NSUB * CHUNKS * K
N_PAD = 10240
ROWS_PER_SUB = N_PAD // NSUB
DUMP_ROW = N + 64
EB = 1024
RB = 1024
GRID_E = E_PAD // EB
GRID_N = N_PAD // RB


def _sc_body_factory(tasks):
    ntask = len(tasks)

    def body(*refs):
        tables = refs[:ntask]
        src4 = refs[ntask]
        dst4 = refs[ntask + 1]
        zblk = refs[ntask + 2]
        outs = refs[ntask + 3:ntask + 3 + ntask]
        acc, src_all, dst_all, rows, rows2, sem, sem2 = refs[ntask + 3 + ntask:]

        c = lax.axis_index("c")
        s = lax.axis_index("s")
        r0 = s * ROWS_PER_SUB

        pltpu.sync_copy(src4.at[c, s], src_all)
        pltpu.sync_copy(dst4.at[c, s], dst_all)

        for t, mode in enumerate(tasks):
            tbl = tables[t]
            out = outs[t]
            pltpu.sync_copy(zblk, acc.at[pl.ds(r0, ROWS_PER_SUB)])
            plsc.subcore_barrier()

            def gather(j, buf, semx):
                if mode == "linear":
                    return pltpu.async_copy(tbl.at[c, s, j], buf, semx)
                return pltpu.async_copy(tbl.at[src_all.at[j]], buf, semx)

            gather(0, rows, sem).wait()

            def pair(jj, carry):
                j = 2 * jj
                cp_b = gather(j + 1, rows2, sem2)
                pltpu.sync_copy(rows, acc.at[dst_all.at[j]], add=True)
                cp_b.wait()
                jn = jnp.minimum(j + 2, CHUNKS - 1)
                cp_a = gather(jn, rows, sem)
                pltpu.sync_copy(rows2, acc.at[dst_all.at[j + 1]], add=True)
                cp_a.wait()
                return carry

            lax.fori_loop(0, CHUNKS // 2, pair, 0)
            plsc.subcore_barrier()
            pltpu.sync_copy(acc.at[pl.ds(r0, ROWS_PER_SUB)],
                            out.at[c, pl.ds(r0, ROWS_PER_SUB)])
            plsc.subcore_barrier()

    return body


def _make_seg_sum(tasks):
    mesh = plsc.VectorSubcoreMesh(core_axis_name="c", subcore_axis_name="s")
    ntask = len(tasks)
    out_type = [jax.ShapeDtypeStruct((NCORE, N_PAD, 128), jnp.float32)
                for _ in range(ntask)]
    scratch = [
        pltpu.VMEM_SHARED((N_PAD, 128), jnp.float32),
        pltpu.VMEM((CHUNKS, K), jnp.int32),
        pltpu.VMEM((CHUNKS, K), jnp.int32),
        pltpu.VMEM((K, 128), jnp.float32),
        pltpu.VMEM((K, 128), jnp.float32),
        pltpu.SemaphoreType.DMA,
        pltpu.SemaphoreType.DMA,
    ]
    return pl.kernel(
        _sc_body_factory(tasks),
        out_type=out_type,
        mesh=mesh,
        scratch_types=scratch,
    )


def _feat_body(t_ref, a_ref, d_ref, embt_ref, embd_ref, wrow_ref, brow_ref,
               metmask_ref, cnt_ref, f_ref):
    t = t_ref[...]
    a = a_ref[...]
    d = d_ref[...]
    ia = lax.broadcasted_iota(jnp.int32, (1, 16), 1)
    idd = lax.broadcasted_iota(jnp.int32, (1, 8), 1)
    oh_a = (a == ia).astype(jnp.float32)
    oh_d = (d == idd).astype(jnp.float32)
    ef = jnp.dot(oh_a, embt_ref[...], preferred_element_type=jnp.float32)
    ef = ef + jnp.dot(oh_d, embd_ref[...], preferred_element_type=jnp.float32)
    et = jnp.cos(t * wrow_ref[...] + brow_ref[...]) * metmask_ref[...]
    f_ref[...] = ef + et + cnt_ref[...]


def _dense0_body(x_ref, pf_ref, px0_ref, px1_ref, wh0_ref, wh1_ref, wf_ref,
                 wr_ref, bias_ref, out_ref, s_ref, ss_ref):
    i = pl.program_id(0)
    sf = pf_ref[0] + pf_ref[1]
    sx0 = px0_ref[0] + px0_ref[1]
    sx1 = px1_ref[0] + px1_ref[1]
    inv = 1.0 / jnp.maximum(sf[:, 127:128], 1.0)
    out = jnp.dot(sx0 * inv, wh0_ref[...], preferred_element_type=jnp.float32)
    out += jnp.dot(sx1 * inv, wh1_ref[...], preferred_element_type=jnp.float32)
    out += jnp.dot(sf * inv, wf_ref[...], preferred_element_type=jnp.float32)
    out += jnp.dot(x_ref[...], wr_ref[...], preferred_element_type=jnp.float32)
    out += bias_ref[...]
    out_ref[...] = out
    ridx = i * RB + lax.broadcasted_iota(jnp.int32, (RB, 1), 0)
    valid = (ridx < N).astype(jnp.float32)
    ov = out * valid

    @pl.when(i == 0)
    def _():
        s_ref[...] = jnp.zeros_like(s_ref)
        ss_ref[...] = jnp.zeros_like(ss_ref)

    s_ref[...] += jnp.sum(ov, axis=0, keepdims=True)
    ss_ref[...] += jnp.sum(ov * ov, axis=0, keepdims=True)


def _bn_elu_p_body(out_ref, s_ref, ss_ref, g_ref, b_ref, w1h_ref,
                   h_ref, p_ref):
    m = s_ref[...] / N
    v = ss_ref[...] / N - m * m
    xn = (out_ref[...] - m) * lax.rsqrt(v + 1e-5) * g_ref[...] + b_ref[...]
    h = jnp.where(xn > 0, xn, jnp.exp(xn) - 1.0)
    h_ref[...] = h
    p_ref[...] = jnp.dot(h, w1h_ref[...], preferred_element_type=jnp.float32)


def _dense1_body(h_ref, pf_ref, pp_ref, wf1_ref, wr1_ref, bias_ref,
                 out_ref, s_ref, ss_ref):
    i = pl.program_id(0)
    sf = pf_ref[0] + pf_ref[1]
    sp = pp_ref[0] + pp_ref[1]
    inv = 1.0 / jnp.maximum(sf[:, 127:128], 1.0)
    out = sp * inv
    out += jnp.dot(sf * inv, wf1_ref[...], preferred_element_type=jnp.float32)
    out += jnp.dot(h_ref[...], wr1_ref[...], preferred_element_type=jnp.float32)
    out += bias_ref[...]
    out_ref[...] = out
    ridx = i * RB + lax.broadcasted_iota(jnp.int32, (RB, 1), 0)
    valid = (ridx < N).astype(jnp.float32)
    ov = out * valid

    @pl.when(i == 0)
    def _():
        s_ref[...] = jnp.zeros_like(s_ref)
        ss_ref[...] = jnp.zeros_like(ss_ref)

    s_ref[...] += jnp.sum(ov, axis=0, keepdims=True)
    ss_ref[...] += jnp.sum(ov * ov, axis=0, keepdims=True)


def _bn_elu_lsm_body(out_ref, s_ref, ss_ref, g_ref, b_ref, y_ref):
    m = s_ref[...] / N
    v = ss_ref[...] / N - m * m
    xn = (out_ref[...] - m) * lax.rsqrt(v + 1e-5) * g_ref[...] + b_ref[...]
    h = jnp.where(xn > 0, xn, jnp.exp(xn) - 1.0)
    mx = jnp.max(h, axis=1, keepdims=True)
    z = h - mx
    lse = jnp.log(jnp.sum(jnp.exp(z), axis=1, keepdims=True))
    y_ref[...] = z - lse


def kernel(x, edge_index, edge_attr, edge_t, edge_d, emb_type, emb_dir, t_w,
           t_b, l0_wl, l0_bl, l0_wr, l0_br, g0, b0, l1_wl, l1_bl, l1_wr,
           l1_br, g1, b1):
    f32 = jnp.float32

    src = edge_index[0].astype(jnp.int32)
    dst = edge_index[1].astype(jnp.int32)
    npad_e = E_PAD - E
    src4 = jnp.concatenate([src, jnp.zeros((npad_e,), jnp.int32)]) \
        .reshape(NCORE, NSUB, CHUNKS, K)
    dump = N + jnp.arange(npad_e, dtype=jnp.int32) % (N_PAD - N)
    dst4 = jnp.concatenate([dst, dump]).reshape(NCORE, NSUB, CHUNKS, K)
    t_pad = jnp.concatenate([edge_t.astype(f32), jnp.zeros((npad_e,), f32)]) \
        .reshape(E_PAD, 1)
    a_pad = jnp.concatenate([edge_attr.astype(jnp.int32),
                             jnp.zeros((npad_e,), jnp.int32)]).reshape(E_PAD, 1)
    d_pad = jnp.concatenate([edge_d.astype(jnp.int32),
                             jnp.zeros((npad_e,), jnp.int32)]).reshape(E_PAD, 1)

    xp = jnp.concatenate([x.astype(f32), jnp.zeros((N_PAD - N, DIN), f32)])
    x0 = xp[:, :128]
    x1 = xp[:, 128:]

    embt = jnp.zeros((16, 128), f32).at[:12, :DE].set(emb_type.astype(f32))
    embd = jnp.zeros((8, 128), f32).at[:2, :DE].set(emb_dir.astype(f32))
    wrow = jnp.zeros((1, 128), f32).at[0, DE:DE + DT].set(t_w.astype(f32))
    brow = jnp.zeros((1, 128), f32).at[0, DE:DE + DT].set(t_b.astype(f32))
    metmask = jnp.zeros((1, 128), f32).at[0, DE:DE + DT].set(1.0)
    cntrow = jnp.zeros((1, 128), f32).at[0, 127].set(1.0)

    wh0 = l0_wl[:, 0:128].T.astype(f32)
    wh1 = l0_wl[:, 128:256].T.astype(f32)
    wf0 = jnp.zeros((128, DH), f32) \
        .at[:DE].set(l0_wl[:, DIN:DIN + DE].T) \
        .at[DE:DE + DT].set(l0_wl[:, DIN + DE:DIN + DE + DT].T)
    wr0 = l0_wr.T.astype(f32)
    bias0 = (l0_bl + l0_br).reshape(1, DH).astype(f32)
    w1h = l1_wl[:, :DH].T.astype(f32)
    wf1 = jnp.zeros((128, DOUT), f32) \
        .at[:DE].set(l1_wl[:, DH:DH + DE].T) \
        .at[DE:DE + DT].set(l1_wl[:, DH + DE:DH + DE + DT].T)
    wr1 = l1_wr.T.astype(f32)
    bias1 = (l1_bl + l1_br).reshape(1, DOUT).astype(f32)
    g0r = g0.reshape(1, DH).astype(f32)
    b0r = b0.reshape(1, DH).astype(f32)
    g1r = g1.reshape(1, DOUT).astype(f32)
    b1r = b1.reshape(1, DOUT).astype(f32)

    zblk = jnp.zeros((ROWS_PER_SUB, 128), f32)

    full = lambda shape: pl.BlockSpec(shape, lambda i: (0,) * len(shape))
    feat = pl.pallas_call(
        _feat_body,
        grid=(GRID_E,),
        in_specs=[
            pl.BlockSpec((EB, 1), lambda i: (i, 0)),
            pl.BlockSpec((EB, 1), lambda i: (i, 0)),
            pl.BlockSpec((EB, 1), lambda i: (i, 0)),
            full((16, 128)), full((8, 128)), full((1, 128)),
            full((1, 128)), full((1, 128)), full((1, 128)),
        ],
        out_specs=pl.BlockSpec((EB, 128), lambda i: (i, 0)),
        out_shape=jax.ShapeDtypeStruct((E_PAD, 128), f32),
    )
    F = feat(t_pad, a_pad, d_pad, embt, embd, wrow, brow, metmask, cntrow)
    F5 = F.reshape(NCORE, NSUB, CHUNKS, K, 128)

    seg3 = _make_seg_sum(("linear", "gather", "gather"))
    pF, pX0, pX1 = seg3(F5, x0, x1, src4, dst4, zblk)

    dense0 = pl.pallas_call(
        _dense0_body,
        grid=(GRID_N,),
        in_specs=[
            pl.BlockSpec((RB, DIN), lambda i: (i, 0)),
            pl.BlockSpec((NCORE, RB, 128), lambda i: (0, i, 0)),
            pl.BlockSpec((NCORE, RB, 128), lambda i: (0, i, 0)),
            pl.BlockSpec((NCORE, RB, 128), lambda i: (0, i, 0)),
            full((128, DH)), full((128, DH)), full((128, DH)),
            full((DIN, DH)), full((1, DH)),
        ],
        out_specs=[
            pl.BlockSpec((RB, DH), lambda i: (i, 0)),
            pl.BlockSpec((1, DH), lambda i: (0, 0)),
            pl.BlockSpec((1, DH), lambda i: (0, 0)),
        ],
        out_shape=[
            jax.ShapeDtypeStruct((N_PAD, DH), f32),
            jax.ShapeDtypeStruct((1, DH), f32),
            jax.ShapeDtypeStruct((1, DH), f32),
        ],
    )
    out0, s0, ss0 = dense0(xp, pF, pX0, pX1, wh0, wh1, wf0, wr0, bias0)

    bn0 = pl.pallas_call(
        _bn_elu_p_body,
        grid=(GRID_N,),
        in_specs=[
            pl.BlockSpec((RB, DH), lambda i: (i, 0)),
            full((1, DH)), full((1, DH)), full((1, DH)), full((1, DH)),
            full((DH, DOUT)),
        ],
        out_specs=[
            pl.BlockSpec((RB, DH), lambda i: (i, 0)),
            pl.BlockSpec((RB, DOUT), lambda i: (i, 0)),
        ],
        out_shape=[
            jax.ShapeDtypeStruct((N_PAD, DH), f32),
            jax.ShapeDtypeStruct((N_PAD, DOUT), f32),
        ],
    )
    h, p = bn0(out0, s0, ss0, g0r, b0r, w1h)

    seg1 = _make_seg_sum(("gather",))
    (pP,) = seg1(p, src4, dst4, zblk)

    dense1 = pl.pallas_call(
        _dense1_body,
        grid=(GRID_N,),
        in_specs=[
            pl.BlockSpec((RB, DH), lambda i: (i, 0)),
            pl.BlockSpec((NCORE, RB, 128), lambda i: (0, i, 0)),
            pl.BlockSpec((NCORE, RB, 128), lambda i: (0, i, 0)),
            full((128, DOUT)), full((DH, DOUT)), full((1, DOUT)),
        ],
        out_specs=[
            pl.BlockSpec((RB, DOUT), lambda i: (i, 0)),
            pl.BlockSpec((1, DOUT), lambda i: (0, 0)),
            pl.BlockSpec((1, DOUT), lambda i: (0, 0)),
        ],
        out_shape=[
            jax.ShapeDtypeStruct((N_PAD, DOUT), f32),
            jax.ShapeDtypeStruct((1, DOUT), f32),
            jax.ShapeDtypeStruct((1, DOUT), f32),
        ],
    )
    out1, s1, ss1 = dense1(h, pF, pP, wf1, wr1, bias1)

    bn1 = pl.pallas_call(
        _bn_elu_lsm_body,
        grid=(GRID_N,),
        in_specs=[
            pl.BlockSpec((RB, DOUT), lambda i: (i, 0)),
            full((1, DOUT)), full((1, DOUT)), full((1, DOUT)), full((1, DOUT)),
        ],
        out_specs=pl.BlockSpec((RB, DOUT), lambda i: (i, 0)),
        out_shape=jax.ShapeDtypeStruct((N_PAD, DOUT), f32),
    )
    y = bn1(out1, s1, ss1, g1r, b1r)
    return y[:N]


# speedup vs baseline: 6.9251x; 2.0047x over previous
import jax
import jax.numpy as jnp
from jax import lax
from jax.experimental import pallas as pl
from jax.experimental.pallas import tpu as pltpu
from jax.experimental.pallas import tpu_sc as plsc

N = 10000
E = 160000
DIN = 256
DH = 512
DOUT = 128
DE = 50
DT = 50

NCORE = 2
NSUB = 16
K = 128
CHUNKS = 40
E_PAD = NCORE * NSUB * CHUNKS * K
N_PAD = 10240
ROWS_PER_SUB = N_PAD // NSUB
EB = 1024
RB = 1000
GRID_E = E_PAD // EB
GRID_N = N // RB


def _sc_body_factory(tasks):
    ntask = len(tasks)

    ngather = sum(1 for m in tasks if m == "gather")

    def body(*refs):
        tables = refs[:ntask]
        srcs = refs[ntask:ntask + ngather]
        dst4 = refs[ntask + ngather]
        zblk = refs[ntask + ngather + 1]
        o0 = ntask + ngather + 2
        outs = refs[o0:o0 + ntask]
        acc, src_all, dst_all, rows, rows2, sem, sem2 = refs[o0 + ntask:]

        c = lax.axis_index("c")
        s = lax.axis_index("s")
        r0 = s * ROWS_PER_SUB

        pltpu.sync_copy(dst4.at[c, s], dst_all)

        gi = 0
        for t, mode in enumerate(tasks):
            tbl = tables[t]
            out = outs[t]
            if mode == "gather":
                pltpu.sync_copy(srcs[gi].at[c, s], src_all)
                gi += 1
            pltpu.sync_copy(zblk, acc.at[pl.ds(r0, ROWS_PER_SUB)])
            plsc.subcore_barrier()

            def gather(j, buf, semx):
                if mode == "linear":
                    return pltpu.async_copy(tbl.at[c, s, j], buf, semx)
                return pltpu.async_copy(tbl.at[src_all.at[j]], buf, semx)

            gather(0, rows, sem).wait()

            def pair(jj, carry):
                j = 2 * jj
                cp_b = gather(j + 1, rows2, sem2)
                pltpu.sync_copy(rows, acc.at[dst_all.at[j]], add=True)
                cp_b.wait()
                jn = jnp.minimum(j + 2, CHUNKS - 1)
                cp_a = gather(jn, rows, sem)
                pltpu.sync_copy(rows2, acc.at[dst_all.at[j + 1]], add=True)
                cp_a.wait()
                return carry

            lax.fori_loop(0, CHUNKS // 2, pair, 0)
            plsc.subcore_barrier()
            pltpu.sync_copy(acc.at[pl.ds(r0, ROWS_PER_SUB)],
                            out.at[c, pl.ds(r0, ROWS_PER_SUB)])
            plsc.subcore_barrier()

    return body


def _make_seg_sum(tasks):
    mesh = plsc.VectorSubcoreMesh(core_axis_name="c", subcore_axis_name="s")
    ntask = len(tasks)
    out_type = [jax.ShapeDtypeStruct((NCORE, N_PAD, 128), jnp.float32)
                for _ in range(ntask)]
    scratch = [
        pltpu.VMEM_SHARED((N_PAD, 128), jnp.float32),
        pltpu.VMEM((CHUNKS, K), jnp.int32),
        pltpu.VMEM((CHUNKS, K), jnp.int32),
        pltpu.VMEM((K, 128), jnp.float32),
        pltpu.VMEM((K, 128), jnp.float32),
        pltpu.SemaphoreType.DMA,
        pltpu.SemaphoreType.DMA,
    ]
    return pl.kernel(
        _sc_body_factory(tasks),
        out_type=out_type,
        mesh=mesh,
        scratch_types=scratch,
    )


def _feat_body(t_ref, a_ref, d_ref, comb_ref, wrow_ref, brow_ref,
               metmask_ref, cnt_ref, f_ref):
    t = t_ref[0]
    idx = a_ref[0] * 2 + d_ref[0]
    io = lax.broadcasted_iota(jnp.int32, (32, 1), 0)
    oh = (idx == io).astype(jnp.float32)
    dn = (((0,), (0,)), ((), ()))
    ef = lax.dot_general(oh, comb_ref[...], dn,
                         preferred_element_type=jnp.float32)
    outer = lax.dot_general(t, wrow_ref[...], dn,
                            preferred_element_type=jnp.float32)
    et = jnp.cos(outer + brow_ref[...]) * metmask_ref[...]
    f_ref[...] = ef + et + cnt_ref[...]


def _dense0_body(x_ref, pf_ref, px0_ref, px1_ref, wh0_ref, wh1_ref, wf_ref,
                 wr_ref, bias_ref, out_ref, s_ref, ss_ref):
    i = pl.program_id(0)
    sf = pf_ref[0] + pf_ref[1]
    sx0 = px0_ref[0] + px0_ref[1]
    sx1 = px1_ref[0] + px1_ref[1]
    inv = 1.0 / jnp.maximum(sf[:, 127:128], 1.0)
    out = jnp.dot(sx0 * inv, wh0_ref[...], preferred_element_type=jnp.float32)
    out += jnp.dot(sx1 * inv, wh1_ref[...], preferred_element_type=jnp.float32)
    out += jnp.dot(sf * inv, wf_ref[...], preferred_element_type=jnp.float32)
    out += jnp.dot(x_ref[...], wr_ref[...], preferred_element_type=jnp.float32)
    out += bias_ref[...]
    out_ref[...] = out

    @pl.when(i == 0)
    def _():
        s_ref[...] = jnp.zeros_like(s_ref)
        ss_ref[...] = jnp.zeros_like(ss_ref)

    s_ref[...] += jnp.sum(out, axis=0, keepdims=True)
    ss_ref[...] += jnp.sum(out * out, axis=0, keepdims=True)


def _bn_elu_p_body(out_ref, s_ref, ss_ref, g_ref, b_ref, w1h_ref,
                   h_ref, p_ref):
    m = s_ref[...] / N
    v = ss_ref[...] / N - m * m
    xn = (out_ref[...] - m) * lax.rsqrt(v + 1e-5) * g_ref[...] + b_ref[...]
    h = jnp.where(xn > 0, xn, jnp.exp(xn) - 1.0)
    h_ref[...] = h
    p_ref[...] = jnp.dot(h, w1h_ref[...], preferred_element_type=jnp.float32)


def _dense1_body(h_ref, pf_ref, pp_ref, wf1_ref, wr1_ref, bias_ref,
                 out_ref, s_ref, ss_ref):
    i = pl.program_id(0)
    sf = pf_ref[0] + pf_ref[1]
    sp = pp_ref[0] + pp_ref[1]
    inv = 1.0 / jnp.maximum(sf[:, 127:128], 1.0)
    out = sp * inv
    out += jnp.dot(sf * inv, wf1_ref[...], preferred_element_type=jnp.float32)
    out += jnp.dot(h_ref[...], wr1_ref[...], preferred_element_type=jnp.float32)
    out += bias_ref[...]
    out_ref[...] = out

    @pl.when(i == 0)
    def _():
        s_ref[...] = jnp.zeros_like(s_ref)
        ss_ref[...] = jnp.zeros_like(ss_ref)

    s_ref[...] += jnp.sum(out, axis=0, keepdims=True)
    ss_ref[...] += jnp.sum(out * out, axis=0, keepdims=True)


def _bn_elu_lsm_body(out_ref, s_ref, ss_ref, g_ref, b_ref, y_ref):
    m = s_ref[...] / N
    v = ss_ref[...] / N - m * m
    xn = (out_ref[...] - m) * lax.rsqrt(v + 1e-5) * g_ref[...] + b_ref[...]
    h = jnp.where(xn > 0, xn, jnp.exp(xn) - 1.0)
    mx = jnp.max(h, axis=1, keepdims=True)
    z = h - mx
    lse = jnp.log(jnp.sum(jnp.exp(z), axis=1, keepdims=True))
    y_ref[...] = z - lse


def kernel(x, edge_index, edge_attr, edge_t, edge_d, emb_type, emb_dir, t_w,
           t_b, l0_wl, l0_bl, l0_wr, l0_br, g0, b0, l1_wl, l1_bl, l1_wr,
           l1_br, g1, b1):
    f32 = jnp.float32

    src = edge_index[0].astype(jnp.int32)
    dst = edge_index[1].astype(jnp.int32)
    nw = NCORE * NSUB
    per_w = E // nw
    pad_w = CHUNKS * K - per_w
    pad_src = (jnp.arange(nw * pad_w, dtype=jnp.int32) * 83) % N
    pad_dst = N + jnp.arange(nw * pad_w, dtype=jnp.int32) % (N_PAD - N)

    def interleave(real, pad):
        return jnp.concatenate(
            [real.reshape(nw, per_w), pad.reshape(nw, pad_w)], axis=1
        ).reshape(NCORE, NSUB, CHUNKS, K)

    src4 = interleave(src, pad_src)
    dst4 = interleave(dst, pad_dst)
    src_even = src4 * 2
    src_odd = src4 * 2 + 1
    zpad_f = jnp.zeros((nw * pad_w,), f32)
    zpad_i = jnp.zeros((nw * pad_w,), jnp.int32)
    t_pad = interleave(edge_t.astype(f32), zpad_f).reshape(GRID_E, 1, EB)
    a_pad = interleave(edge_attr.astype(jnp.int32), zpad_i) \
        .reshape(GRID_E, 1, EB)
    d_pad = interleave(edge_d.astype(jnp.int32), zpad_i) \
        .reshape(GRID_E, 1, EB)

    x2 = x.astype(f32).reshape(2 * N, 128)

    comb = jnp.zeros((32, 128), f32).at[:24, :DE].set(
        (emb_type[:, None, :] + emb_dir[None, :, :]).reshape(24, DE)
    )
    wrow = jnp.zeros((1, 128), f32).at[0, DE:DE + DT].set(t_w.astype(f32))
    brow = jnp.zeros((1, 128), f32).at[0, DE:DE + DT].set(t_b.astype(f32))
    metmask = jnp.zeros((1, 128), f32).at[0, DE:DE + DT].set(1.0)
    cntrow = jnp.zeros((1, 128), f32).at[0, 127].set(1.0)

    wh0 = l0_wl[:, 0:128].T.astype(f32)
    wh1 = l0_wl[:, 128:256].T.astype(f32)
    wf0 = jnp.zeros((128, DH), f32) \
        .at[:DE].set(l0_wl[:, DIN:DIN + DE].T) \
        .at[DE:DE + DT].set(l0_wl[:, DIN + DE:DIN + DE + DT].T)
    wr0 = l0_wr.T.astype(f32)
    bias0 = (l0_bl + l0_br).reshape(1, DH).astype(f32)
    w1h = l1_wl[:, :DH].T.astype(f32)
    wf1 = jnp.zeros((128, DOUT), f32) \
        .at[:DE].set(l1_wl[:, DH:DH + DE].T) \
        .at[DE:DE + DT].set(l1_wl[:, DH + DE:DH + DE + DT].T)
    wr1 = l1_wr.T.astype(f32)
    bias1 = (l1_bl + l1_br).reshape(1, DOUT).astype(f32)
    g0r = g0.reshape(1, DH).astype(f32)
    b0r = b0.reshape(1, DH).astype(f32)
    g1r = g1.reshape(1, DOUT).astype(f32)
    b1r = b1.reshape(1, DOUT).astype(f32)

    zblk = jnp.zeros((ROWS_PER_SUB, 128), f32)

    full = lambda shape: pl.BlockSpec(shape, lambda i: (0,) * len(shape))
    feat = pl.pallas_call(
        _feat_body,
        grid=(GRID_E,),
        in_specs=[
            pl.BlockSpec((1, 1, EB), lambda i: (i, 0, 0)),
            pl.BlockSpec((1, 1, EB), lambda i: (i, 0, 0)),
            pl.BlockSpec((1, 1, EB), lambda i: (i, 0, 0)),
            full((32, 128)), full((1, 128)),
            full((1, 128)), full((1, 128)), full((1, 128)),
        ],
        out_specs=pl.BlockSpec((EB, 128), lambda i: (i, 0)),
        out_shape=jax.ShapeDtypeStruct((E_PAD, 128), f32),
    )
    F = feat(t_pad, a_pad, d_pad, comb, wrow, brow, metmask, cntrow)
    F5 = F.reshape(NCORE, NSUB, CHUNKS, K, 128)

    seg3 = _make_seg_sum(("linear", "gather", "gather"))
    pF, pX0, pX1 = seg3(F5, x2, x2, src_even, src_odd, dst4, zblk)

    dense0 = pl.pallas_call(
        _dense0_body,
        grid=(GRID_N,),
        in_specs=[
            pl.BlockSpec((RB, DIN), lambda i: (i, 0)),
            pl.BlockSpec((NCORE, RB, 128), lambda i: (0, i, 0)),
            pl.BlockSpec((NCORE, RB, 128), lambda i: (0, i, 0)),
            pl.BlockSpec((NCORE, RB, 128), lambda i: (0, i, 0)),
            full((128, DH)), full((128, DH)), full((128, DH)),
            full((DIN, DH)), full((1, DH)),
        ],
        out_specs=[
            pl.BlockSpec((RB, DH), lambda i: (i, 0)),
            pl.BlockSpec((1, DH), lambda i: (0, 0)),
            pl.BlockSpec((1, DH), lambda i: (0, 0)),
        ],
        out_shape=[
            jax.ShapeDtypeStruct((N, DH), f32),
            jax.ShapeDtypeStruct((1, DH), f32),
            jax.ShapeDtypeStruct((1, DH), f32),
        ],
    )
    out0, s0, ss0 = dense0(x.astype(jnp.float32), pF, pX0, pX1, wh0, wh1,
                           wf0, wr0, bias0)

    bn0 = pl.pallas_call(
        _bn_elu_p_body,
        grid=(GRID_N,),
        in_specs=[
            pl.BlockSpec((RB, DH), lambda i: (i, 0)),
            full((1, DH)), full((1, DH)), full((1, DH)), full((1, DH)),
            full((DH, DOUT)),
        ],
        out_specs=[
            pl.BlockSpec((RB, DH), lambda i: (i, 0)),
            pl.BlockSpec((RB, DOUT), lambda i: (i, 0)),
        ],
        out_shape=[
            jax.ShapeDtypeStruct((N, DH), f32),
            jax.ShapeDtypeStruct((N, DOUT), f32),
        ],
    )
    h, p = bn0(out0, s0, ss0, g0r, b0r, w1h)

    seg1 = _make_seg_sum(("gather",))
    (pP,) = seg1(p, src4, dst4, zblk)

    dense1 = pl.pallas_call(
        _dense1_body,
        grid=(GRID_N,),
        in_specs=[
            pl.BlockSpec((RB, DH), lambda i: (i, 0)),
            pl.BlockSpec((NCORE, RB, 128), lambda i: (0, i, 0)),
            pl.BlockSpec((NCORE, RB, 128), lambda i: (0, i, 0)),
            full((128, DOUT)), full((DH, DOUT)), full((1, DOUT)),
        ],
        out_specs=[
            pl.BlockSpec((RB, DOUT), lambda i: (i, 0)),
            pl.BlockSpec((1, DOUT), lambda i: (0, 0)),
            pl.BlockSpec((1, DOUT), lambda i: (0, 0)),
        ],
        out_shape=[
            jax.ShapeDtypeStruct((N, DOUT), f32),
            jax.ShapeDtypeStruct((1, DOUT), f32),
            jax.ShapeDtypeStruct((1, DOUT), f32),
        ],
    )
    out1, s1, ss1 = dense1(h, pF, pP, wf1, wr1, bias1)

    bn1 = pl.pallas_call(
        _bn_elu_lsm_body,
        grid=(GRID_N,),
        in_specs=[
            pl.BlockSpec((RB, DOUT), lambda i: (i, 0)),
            full((1, DOUT)), full((1, DOUT)), full((1, DOUT)), full((1, DOUT)),
        ],
        out_specs=pl.BlockSpec((RB, DOUT), lambda i: (i, 0)),
        out_shape=jax.ShapeDtypeStruct((N, DOUT), f32),
    )
    return bn1(out1, s1, ss1, g1r, b1r)


# speedup vs baseline: 9.1453x; 1.3206x over previous
import jax
import jax.numpy as jnp
from jax import lax
from jax.experimental import pallas as pl
from jax.experimental.pallas import tpu as pltpu
from jax.experimental.pallas import tpu_sc as plsc

N = 10000
E = 160000
DIN = 256
DH = 512
DOUT = 128
DE = 50
DT = 50

NCORE = 2
NSUB = 16
K = 128
CHUNKS = 40
E_PAD = NCORE * NSUB * CHUNKS * K
N_PAD = 10240
ROWS_PER_SUB = N_PAD // NSUB
EB = 1024
RB = 1000
GRID_E = E_PAD // EB
GRID_N = N // RB


def _scx_body(x2, se4, so4, dst4, zblk,
              pX0, pX1,
              acc, idx_all, dst_all, rows, rows2, sem, sem2):
    c = lax.axis_index("c")
    s = lax.axis_index("s")
    r0 = s * ROWS_PER_SUB
    pltpu.sync_copy(dst4.at[c, s], dst_all)

    def seg_task(src4, out_ref):
        pltpu.sync_copy(src4.at[c, s], idx_all)
        pltpu.sync_copy(zblk, acc.at[pl.ds(r0, ROWS_PER_SUB)])
        plsc.subcore_barrier()

        def gather(j, b, sx):
            return pltpu.async_copy(x2.at[idx_all.at[j]], b, sx)

        gather(0, rows, sem).wait()

        def pair(jj, carry):
            j = 2 * jj
            cpb = gather(j + 1, rows2, sem2)
            pltpu.sync_copy(rows, acc.at[dst_all.at[j]], add=True)
            cpb.wait()
            jn = jnp.minimum(j + 2, CHUNKS - 1)
            cpa = gather(jn, rows, sem)
            pltpu.sync_copy(rows2, acc.at[dst_all.at[j + 1]], add=True)
            cpa.wait()
            return carry

        lax.fori_loop(0, CHUNKS // 2, pair, 0)
        plsc.subcore_barrier()
        pltpu.sync_copy(acc.at[pl.ds(r0, ROWS_PER_SUB)],
                        out_ref.at[c, pl.ds(r0, ROWS_PER_SUB)])
        plsc.subcore_barrier()

    seg_task(se4, pX0)
    seg_task(so4, pX1)


def _scf_body(f5, dst4, z32, pF,
              acc32, dst_all, r32, r32b, sem, sem2):
    c = lax.axis_index("c")
    s = lax.axis_index("s")
    r0 = s * ROWS_PER_SUB
    pltpu.sync_copy(dst4.at[c, s], dst_all)
    pltpu.sync_copy(z32, acc32.at[pl.ds(r0, ROWS_PER_SUB)])
    plsc.subcore_barrier()

    def gather(j, b, sx):
        return pltpu.async_copy(f5.at[c, s, j], b, sx)

    gather(0, r32, sem).wait()

    def pair(jj, carry):
        j = 2 * jj
        cpb = gather(j + 1, r32b, sem2)
        pltpu.sync_copy(r32, acc32.at[dst_all.at[j]], add=True)
        cpb.wait()
        jn = jnp.minimum(j + 2, CHUNKS - 1)
        cpa = gather(jn, r32, sem)
        pltpu.sync_copy(r32b, acc32.at[dst_all.at[j + 1]], add=True)
        cpa.wait()
        return carry

    lax.fori_loop(0, CHUNKS // 2, pair, 0)
    plsc.subcore_barrier()
    pltpu.sync_copy(acc32.at[pl.ds(r0, ROWS_PER_SUB)],
                    pF.at[c, pl.ds(r0, ROWS_PER_SUB)])
    plsc.subcore_barrier()


def _sc2_body(p2, sr4, dst4, zblk, pP,
              acc, idx_all, dst_all, rows, rows2, sem, sem2):
    c = lax.axis_index("c")
    s = lax.axis_index("s")
    r0 = s * ROWS_PER_SUB
    pltpu.sync_copy(dst4.at[c, s], dst_all)
    pltpu.sync_copy(sr4.at[c, s], idx_all)
    pltpu.sync_copy(zblk, acc.at[pl.ds(r0, ROWS_PER_SUB)])
    plsc.subcore_barrier()

    def gather(j, b, sx):
        return pltpu.async_copy(p2.at[idx_all.at[j]], b, sx)

    gather(0, rows, sem).wait()

    def pair(jj, carry):
        j = 2 * jj
        cpb = gather(j + 1, rows2, sem2)
        pltpu.sync_copy(rows, acc.at[dst_all.at[j]], add=True)
        cpb.wait()
        jn = jnp.minimum(j + 2, CHUNKS - 1)
        cpa = gather(jn, rows, sem)
        pltpu.sync_copy(rows2, acc.at[dst_all.at[j + 1]], add=True)
        cpa.wait()
        return carry

    lax.fori_loop(0, CHUNKS // 2, pair, 0)
    plsc.subcore_barrier()
    pltpu.sync_copy(acc.at[pl.ds(r0, ROWS_PER_SUB)],
                    pP.at[c, pl.ds(r0, ROWS_PER_SUB)])
    plsc.subcore_barrier()


def _make_scx():
    mesh = plsc.VectorSubcoreMesh(core_axis_name="c", subcore_axis_name="s")
    out_type = [
        jax.ShapeDtypeStruct((NCORE, N_PAD, 128), jnp.float32),
        jax.ShapeDtypeStruct((NCORE, N_PAD, 128), jnp.float32),
    ]
    scratch = [
        pltpu.VMEM_SHARED((N_PAD, 128), jnp.float32),
        pltpu.VMEM((CHUNKS, K), jnp.int32),
        pltpu.VMEM((CHUNKS, K), jnp.int32),
        pltpu.VMEM((K, 128), jnp.float32),
        pltpu.VMEM((K, 128), jnp.float32),
        pltpu.SemaphoreType.DMA,
        pltpu.SemaphoreType.DMA,
    ]
    return pl.kernel(_scx_body, out_type=out_type, mesh=mesh,
                     scratch_types=scratch)


def _make_scf():
    mesh = plsc.VectorSubcoreMesh(core_axis_name="c", subcore_axis_name="s")
    out_type = [jax.ShapeDtypeStruct((NCORE, N_PAD, 32), jnp.float32)]
    scratch = [
        pltpu.VMEM_SHARED((N_PAD, 32), jnp.float32),
        pltpu.VMEM((CHUNKS, K), jnp.int32),
        pltpu.VMEM((K, 32), jnp.float32),
        pltpu.VMEM((K, 32), jnp.float32),
        pltpu.SemaphoreType.DMA,
        pltpu.SemaphoreType.DMA,
    ]
    return pl.kernel(_scf_body, out_type=out_type, mesh=mesh,
                     scratch_types=scratch)


def _make_sc2():
    mesh = plsc.VectorSubcoreMesh(core_axis_name="c", subcore_axis_name="s")
    out_type = [jax.ShapeDtypeStruct((NCORE, N_PAD, 128), jnp.float32)]
    scratch = [
        pltpu.VMEM_SHARED((N_PAD, 128), jnp.float32),
        pltpu.VMEM((CHUNKS, K), jnp.int32),
        pltpu.VMEM((CHUNKS, K), jnp.int32),
        pltpu.VMEM((K, 128), jnp.float32),
        pltpu.VMEM((K, 128), jnp.float32),
        pltpu.SemaphoreType.DMA,
        pltpu.SemaphoreType.DMA,
    ]
    return pl.kernel(_sc2_body, out_type=out_type, mesh=mesh,
                     scratch_types=scratch)


def _feat_body(t_ref, a_ref, d_ref, eye_ref, f_ref):
    t = t_ref[0]
    rows = [jnp.ones_like(t)]
    for _ in range(11):
        rows.append(rows[-1] * t)
    ia = lax.broadcasted_iota(jnp.int32, (12, 1), 0)
    idd = lax.broadcasted_iota(jnp.int32, (2, 1), 0)
    oh_a = (a_ref[0] == ia).astype(jnp.float32)
    oh_d = (d_ref[0] == idd).astype(jnp.float32)
    z = jnp.zeros((6, t.shape[1]), jnp.float32)
    smat = jnp.concatenate(rows + [oh_a, oh_d, z], axis=0)
    dn = (((0,), (0,)), ((), ()))
    f_ref[...] = lax.dot_general(smat, eye_ref[...], dn,
                                 preferred_element_type=jnp.float32)


def _dense0_body(x_ref, pf_ref, px0_ref, px1_ref, wh0_ref, wh1_ref,
                 wf0_ref, wr_ref, bias_ref, out_ref, s_ref, ss_ref):
    i = pl.program_id(0)
    sf = pf_ref[0] + pf_ref[1]
    sx0 = px0_ref[0] + px0_ref[1]
    sx1 = px1_ref[0] + px1_ref[1]
    inv = 1.0 / jnp.maximum(sf[:, 0:1], 1.0)
    out = jnp.dot(sx0 * inv, wh0_ref[...], preferred_element_type=jnp.float32)
    out += jnp.dot(sx1 * inv, wh1_ref[...], preferred_element_type=jnp.float32)
    out += jnp.dot(sf * inv, wf0_ref[...], preferred_element_type=jnp.float32)
    out += jnp.dot(x_ref[...], wr_ref[...], preferred_element_type=jnp.float32)
    out += bias_ref[...]
    out_ref[...] = out

    @pl.when(i == 0)
    def _():
        s_ref[...] = jnp.zeros_like(s_ref)
        ss_ref[...] = jnp.zeros_like(ss_ref)

    s_ref[...] += jnp.sum(out, axis=0, keepdims=True)
    ss_ref[...] += jnp.sum(out * out, axis=0, keepdims=True)


def _bn_elu_p_body(out_ref, s_ref, ss_ref, g_ref, b_ref, w1h_ref,
                   h_ref, p_ref):
    m = s_ref[...] / N
    v = ss_ref[...] / N - m * m
    xn = (out_ref[...] - m) * lax.rsqrt(v + 1e-5) * g_ref[...] + b_ref[...]
    h = jnp.where(xn > 0, xn, jnp.exp(xn) - 1.0)
    h_ref[...] = h
    p_ref[...] = jnp.dot(h, w1h_ref[...], preferred_element_type=jnp.float32)


def _dense1_body(h_ref, pf_ref, pp_ref, wf1_ref, wr1_ref,
                 bias_ref, out_ref, s_ref, ss_ref):
    i = pl.program_id(0)
    sf = pf_ref[0] + pf_ref[1]
    sp = pp_ref[0] + pp_ref[1]
    inv = 1.0 / jnp.maximum(sf[:, 0:1], 1.0)
    out = sp * inv
    out += jnp.dot(sf * inv, wf1_ref[...], preferred_element_type=jnp.float32)
    out += jnp.dot(h_ref[...], wr1_ref[...], preferred_element_type=jnp.float32)
    out += bias_ref[...]
    out_ref[...] = out

    @pl.when(i == 0)
    def _():
        s_ref[...] = jnp.zeros_like(s_ref)
        ss_ref[...] = jnp.zeros_like(ss_ref)

    s_ref[...] += jnp.sum(out, axis=0, keepdims=True)
    ss_ref[...] += jnp.sum(out * out, axis=0, keepdims=True)


def _bn_elu_lsm_body(out_ref, s_ref, ss_ref, g_ref, b_ref, y_ref):
    m = s_ref[...] / N
    v = ss_ref[...] / N - m * m
    xn = (out_ref[...] - m) * lax.rsqrt(v + 1e-5) * g_ref[...] + b_ref[...]
    h = jnp.where(xn > 0, xn, jnp.exp(xn) - 1.0)
    mx = jnp.max(h, axis=1, keepdims=True)
    z = h - mx
    lse = jnp.log(jnp.sum(jnp.exp(z), axis=1, keepdims=True))
    y_ref[...] = z - lse


def kernel(x, edge_index, edge_attr, edge_t, edge_d, emb_type, emb_dir, t_w,
           t_b, l0_wl, l0_bl, l0_wr, l0_br, g0, b0, l1_wl, l1_bl, l1_wr,
           l1_br, g1, b1):
    f32 = jnp.float32

    src = edge_index[0].astype(jnp.int32)
    dst = edge_index[1].astype(jnp.int32)
    nw = NCORE * NSUB
    per_w = E // nw
    pad_w = CHUNKS * K - per_w
    pad_src = (jnp.arange(nw * pad_w, dtype=jnp.int32) * 83) % N
    pad_dst = N + jnp.arange(nw * pad_w, dtype=jnp.int32) % (N_PAD - N)

    def interleave(real, pad):
        return jnp.concatenate(
            [real.reshape(nw, per_w), pad.reshape(nw, pad_w)], axis=1
        ).reshape(NCORE, NSUB, CHUNKS, K)

    src4 = interleave(src, pad_src)
    dst4 = interleave(dst, pad_dst)
    src_even = src4 * 2
    src_odd = src4 * 2 + 1
    zpad_f = jnp.zeros((nw * pad_w,), f32)
    zpad_i = jnp.zeros((nw * pad_w,), jnp.int32)
    t_pad = interleave(edge_t.astype(f32), zpad_f).reshape(GRID_E, 1, EB)
    a_pad = interleave(edge_attr.astype(jnp.int32), zpad_i) \
        .reshape(GRID_E, 1, EB)
    d_pad = interleave(edge_d.astype(jnp.int32), zpad_i) \
        .reshape(GRID_E, 1, EB)

    x2 = x.astype(f32).reshape(2 * N, 128)

    marr = jnp.arange(16, dtype=f32).reshape(16, 1)
    fact = jnp.cumprod(jnp.maximum(jnp.arange(16, dtype=f32), 1.0)) \
        .reshape(16, 1)
    twf = t_w.astype(f32).reshape(1, DT)
    tbf = t_b.astype(f32).reshape(1, DT)
    wpoly = (twf ** marr) * jnp.cos(tbf + marr * (jnp.pi / 2)) / fact
    wpoly = wpoly[:12]

    def wfeat(wl_ef, wl_et, dout):
        w = jnp.zeros((32, dout), f32)
        w = w.at[:12].set(wpoly @ wl_et.T.astype(f32))
        w = w.at[12:24].set(emb_type.astype(f32) @ wl_ef.T.astype(f32))
        w = w.at[24:26].set(emb_dir.astype(f32) @ wl_ef.T.astype(f32))
        return w

    wh0 = l0_wl[:, 0:128].T.astype(f32)
    wh1 = l0_wl[:, 128:256].T.astype(f32)
    wf0 = wfeat(l0_wl[:, DIN:DIN + DE], l0_wl[:, DIN + DE:DIN + DE + DT], DH)
    wr0 = l0_wr.T.astype(f32)
    bias0 = (l0_bl + l0_br).reshape(1, DH).astype(f32)
    w1h = l1_wl[:, :DH].T.astype(f32)
    wf1 = wfeat(l1_wl[:, DH:DH + DE], l1_wl[:, DH + DE:DH + DE + DT], DOUT)
    wr1 = l1_wr.T.astype(f32)
    bias1 = (l1_bl + l1_br).reshape(1, DOUT).astype(f32)
    g0r = g0.reshape(1, DH).astype(f32)
    b0r = b0.reshape(1, DH).astype(f32)
    g1r = g1.reshape(1, DOUT).astype(f32)
    b1r = b1.reshape(1, DOUT).astype(f32)

    zblk = jnp.zeros((ROWS_PER_SUB, 128), f32)
    z32 = jnp.zeros((ROWS_PER_SUB, 32), f32)
    eye32 = jnp.eye(32, dtype=f32)

    full = lambda shape: pl.BlockSpec(shape, lambda i: (0,) * len(shape))
    feat = pl.pallas_call(
        _feat_body,
        grid=(GRID_E,),
        in_specs=[
            pl.BlockSpec((1, 1, EB), lambda i: (i, 0, 0)),
            pl.BlockSpec((1, 1, EB), lambda i: (i, 0, 0)),
            pl.BlockSpec((1, 1, EB), lambda i: (i, 0, 0)),
            full((32, 32)),
        ],
        out_specs=pl.BlockSpec((EB, 32), lambda i: (i, 0)),
        out_shape=jax.ShapeDtypeStruct((E_PAD, 32), f32),
    )
    F = feat(t_pad, a_pad, d_pad, eye32)
    F5 = F.reshape(NCORE, NSUB, CHUNKS, K, 32)

    (pF,) = _make_scf()(F5, dst4, z32)
    pX0, pX1 = _make_scx()(x2, src_even, src_odd, dst4, zblk)

    dense0 = pl.pallas_call(
        _dense0_body,
        grid=(GRID_N,),
        in_specs=[
            pl.BlockSpec((RB, DIN), lambda i: (i, 0)),
            pl.BlockSpec((NCORE, RB, 32), lambda i: (0, i, 0)),
            pl.BlockSpec((NCORE, RB, 128), lambda i: (0, i, 0)),
            pl.BlockSpec((NCORE, RB, 128), lambda i: (0, i, 0)),
            full((128, DH)), full((128, DH)), full((32, DH)),
            full((DIN, DH)), full((1, DH)),
        ],
        out_specs=[
            pl.BlockSpec((RB, DH), lambda i: (i, 0)),
            pl.BlockSpec((1, DH), lambda i: (0, 0)),
            pl.BlockSpec((1, DH), lambda i: (0, 0)),
        ],
        out_shape=[
            jax.ShapeDtypeStruct((N, DH), f32),
            jax.ShapeDtypeStruct((1, DH), f32),
            jax.ShapeDtypeStruct((1, DH), f32),
        ],
    )
    out0, s0, ss0 = dense0(x.astype(f32), pF, pX0, pX1, wh0, wh1,
                           wf0, wr0, bias0)

    bn0 = pl.pallas_call(
        _bn_elu_p_body,
        grid=(GRID_N,),
        in_specs=[
            pl.BlockSpec((RB, DH), lambda i: (i, 0)),
            full((1, DH)), full((1, DH)), full((1, DH)), full((1, DH)),
            full((DH, DOUT)),
        ],
        out_specs=[
            pl.BlockSpec((RB, DH), lambda i: (i, 0)),
            pl.BlockSpec((RB, DOUT), lambda i: (i, 0)),
        ],
        out_shape=[
            jax.ShapeDtypeStruct((N, DH), f32),
            jax.ShapeDtypeStruct((N, DOUT), f32),
        ],
    )
    h, p = bn0(out0, s0, ss0, g0r, b0r, w1h)

    sc2 = _make_sc2()
    (pP,) = sc2(p, src4, dst4, zblk)

    dense1 = pl.pallas_call(
        _dense1_body,
        grid=(GRID_N,),
        in_specs=[
            pl.BlockSpec((RB, DH), lambda i: (i, 0)),
            pl.BlockSpec((NCORE, RB, 32), lambda i: (0, i, 0)),
            pl.BlockSpec((NCORE, RB, 128), lambda i: (0, i, 0)),
            full((32, DOUT)), full((DH, DOUT)), full((1, DOUT)),
        ],
        out_specs=[
            pl.BlockSpec((RB, DOUT), lambda i: (i, 0)),
            pl.BlockSpec((1, DOUT), lambda i: (0, 0)),
            pl.BlockSpec((1, DOUT), lambda i: (0, 0)),
        ],
        out_shape=[
            jax.ShapeDtypeStruct((N, DOUT), f32),
            jax.ShapeDtypeStruct((1, DOUT), f32),
            jax.ShapeDtypeStruct((1, DOUT), f32),
        ],
    )
    out1, s1, ss1 = dense1(h, pF, pP, wf1, wr1, bias1)

    bn1 = pl.pallas_call(
        _bn_elu_lsm_body,
        grid=(GRID_N,),
        in_specs=[
            pl.BlockSpec((RB, DOUT), lambda i: (i, 0)),
            full((1, DOUT)), full((1, DOUT)), full((1, DOUT)), full((1, DOUT)),
        ],
        out_specs=pl.BlockSpec((RB, DOUT), lambda i: (i, 0)),
        out_shape=jax.ShapeDtypeStruct((N, DOUT), f32),
    )
    return bn1(out1, s1, ss1, g1r, b1r)


# speedup vs baseline: 9.9120x; 1.0838x over previous
import jax
import jax.numpy as jnp
from jax import lax
from jax.experimental import pallas as pl
from jax.experimental.pallas import tpu as pltpu
from jax.experimental.pallas import tpu_sc as plsc

N = 10000
E = 160000
DIN = 256
DH = 512
DOUT = 128
DE = 50
DT = 50

NCORE = 2
NSUB = 16
K = 128
CHUNKS = 40
E_PAD = NCORE * NSUB * CHUNKS * K
N_PAD = 10240
ROWS_PER_SUB = N_PAD // NSUB
EB = 1024
RB = 1000
GRID_E = E_PAD // EB
GRID_N = N // RB


K2 = 64
CHUNKS2 = CHUNKS * K // K2
NBUF = 3


def _gather_pipeline(mk, scatter_idx, acc, bufs, sems):
    for b in range(NBUF):
        mk(b, bufs[b], sems[b]).start()

    def step(j, carry):
        b = lax.rem(j, NBUF)
        for bi in range(NBUF):
            @pl.when(b == bi)
            def _():
                mk(0, bufs[bi], sems[bi]).wait()
                pltpu.sync_copy(bufs[bi], acc.at[scatter_idx(j)], add=True)
                mk(j + NBUF, bufs[bi], sems[bi]).start()
        return carry

    lax.fori_loop(0, CHUNKS2 - NBUF, step, 0)
    for k in range(NBUF):
        j = CHUNKS2 - NBUF + k
        bi = j % NBUF
        mk(0, bufs[bi], sems[bi]).wait()
        pltpu.sync_copy(bufs[bi], acc.at[scatter_idx(jnp.int32(j))], add=True)


def _scx_body(x2, se4, so4, dst4, zblk,
              pX0, pX1,
              acc, idx_all, dst_all, b0, b1, b2, s0, s1, s2):
    c = lax.axis_index("c")
    s = lax.axis_index("s")
    r0 = s * ROWS_PER_SUB
    bufs = (b0, b1, b2)
    sems = (s0, s1, s2)
    pltpu.sync_copy(dst4.at[c, s], dst_all)

    def seg_task(src4, out_ref):
        pltpu.sync_copy(src4.at[c, s], idx_all)
        pltpu.sync_copy(zblk, acc.at[pl.ds(r0, ROWS_PER_SUB)])
        plsc.subcore_barrier()
        _gather_pipeline(
            lambda j, b, sx: pltpu.make_async_copy(x2.at[idx_all.at[j]], b, sx),
            lambda j: dst_all.at[j], acc, bufs, sems)
        plsc.subcore_barrier()
        pltpu.sync_copy(acc.at[pl.ds(r0, ROWS_PER_SUB)],
                        out_ref.at[c, pl.ds(r0, ROWS_PER_SUB)])
        plsc.subcore_barrier()

    seg_task(se4, pX0)
    seg_task(so4, pX1)


def _scf_body(f5, dst4, z32, pF,
              acc32, dst_all, r32, r32b, sem, sem2):
    c = lax.axis_index("c")
    s = lax.axis_index("s")
    r0 = s * ROWS_PER_SUB
    pltpu.sync_copy(dst4.at[c, s], dst_all)
    pltpu.sync_copy(z32, acc32.at[pl.ds(r0, ROWS_PER_SUB)])
    plsc.subcore_barrier()

    def gather(j, b, sx):
        return pltpu.async_copy(f5.at[c, s, j], b, sx)

    gather(0, r32, sem).wait()

    def pair(jj, carry):
        j = 2 * jj
        cpb = gather(j + 1, r32b, sem2)
        pltpu.sync_copy(r32, acc32.at[dst_all.at[j]], add=True)
        cpb.wait()
        jn = jnp.minimum(j + 2, CHUNKS - 1)
        cpa = gather(jn, r32, sem)
        pltpu.sync_copy(r32b, acc32.at[dst_all.at[j + 1]], add=True)
        cpa.wait()
        return carry

    lax.fori_loop(0, CHUNKS // 2, pair, 0)
    plsc.subcore_barrier()
    pltpu.sync_copy(acc32.at[pl.ds(r0, ROWS_PER_SUB)],
                    pF.at[c, pl.ds(r0, ROWS_PER_SUB)])
    plsc.subcore_barrier()


def _sc2_body(p2, sr4, dst4, zblk, pP,
              acc, idx_all, dst_all, b0, b1, b2, s0, s1, s2):
    c = lax.axis_index("c")
    s = lax.axis_index("s")
    r0 = s * ROWS_PER_SUB
    bufs = (b0, b1, b2)
    sems = (s0, s1, s2)
    pltpu.sync_copy(dst4.at[c, s], dst_all)
    pltpu.sync_copy(sr4.at[c, s], idx_all)
    pltpu.sync_copy(zblk, acc.at[pl.ds(r0, ROWS_PER_SUB)])
    plsc.subcore_barrier()
    _gather_pipeline(
        lambda j, b, sx: pltpu.make_async_copy(p2.at[idx_all.at[j]], b, sx),
        lambda j: dst_all.at[j], acc, bufs, sems)
    plsc.subcore_barrier()
    pltpu.sync_copy(acc.at[pl.ds(r0, ROWS_PER_SUB)],
                    pP.at[c, pl.ds(r0, ROWS_PER_SUB)])
    plsc.subcore_barrier()


def _make_scx():
    mesh = plsc.VectorSubcoreMesh(core_axis_name="c", subcore_axis_name="s")
    out_type = [
        jax.ShapeDtypeStruct((NCORE, N_PAD, 128), jnp.float32),
        jax.ShapeDtypeStruct((NCORE, N_PAD, 128), jnp.float32),
    ]
    scratch = [
        pltpu.VMEM_SHARED((N_PAD, 128), jnp.float32),
        pltpu.VMEM((CHUNKS2, K2), jnp.int32),
        pltpu.VMEM((CHUNKS2, K2), jnp.int32),
        pltpu.VMEM((K2, 128), jnp.float32),
        pltpu.VMEM((K2, 128), jnp.float32),
        pltpu.VMEM((K2, 128), jnp.float32),
        pltpu.SemaphoreType.DMA,
        pltpu.SemaphoreType.DMA,
        pltpu.SemaphoreType.DMA,
    ]
    return pl.kernel(_scx_body, out_type=out_type, mesh=mesh,
                     scratch_types=scratch)


def _make_scf():
    mesh = plsc.VectorSubcoreMesh(core_axis_name="c", subcore_axis_name="s")
    out_type = [jax.ShapeDtypeStruct((NCORE, N_PAD, 32), jnp.float32)]
    scratch = [
        pltpu.VMEM_SHARED((N_PAD, 32), jnp.float32),
        pltpu.VMEM((CHUNKS, K), jnp.int32),
        pltpu.VMEM((K, 32), jnp.float32),
        pltpu.VMEM((K, 32), jnp.float32),
        pltpu.SemaphoreType.DMA,
        pltpu.SemaphoreType.DMA,
    ]
    return pl.kernel(_scf_body, out_type=out_type, mesh=mesh,
                     scratch_types=scratch)


def _make_sc2():
    mesh = plsc.VectorSubcoreMesh(core_axis_name="c", subcore_axis_name="s")
    out_type = [jax.ShapeDtypeStruct((NCORE, N_PAD, 128), jnp.float32)]
    scratch = [
        pltpu.VMEM_SHARED((N_PAD, 128), jnp.float32),
        pltpu.VMEM((CHUNKS2, K2), jnp.int32),
        pltpu.VMEM((CHUNKS2, K2), jnp.int32),
        pltpu.VMEM((K2, 128), jnp.float32),
        pltpu.VMEM((K2, 128), jnp.float32),
        pltpu.VMEM((K2, 128), jnp.float32),
        pltpu.SemaphoreType.DMA,
        pltpu.SemaphoreType.DMA,
        pltpu.SemaphoreType.DMA,
    ]
    return pl.kernel(_sc2_body, out_type=out_type, mesh=mesh,
                     scratch_types=scratch)


def _feat_body(t_ref, a_ref, d_ref, eye_ref, f_ref):
    t = t_ref[0]
    rows = [jnp.ones_like(t)]
    for _ in range(11):
        rows.append(rows[-1] * t)
    ia = lax.broadcasted_iota(jnp.int32, (12, 1), 0)
    idd = lax.broadcasted_iota(jnp.int32, (2, 1), 0)
    oh_a = (a_ref[0] == ia).astype(jnp.float32)
    oh_d = (d_ref[0] == idd).astype(jnp.float32)
    z = jnp.zeros((6, t.shape[1]), jnp.float32)
    smat = jnp.concatenate(rows + [oh_a, oh_d, z], axis=0)
    dn = (((0,), (0,)), ((), ()))
    f_ref[...] = lax.dot_general(smat, eye_ref[...], dn,
                                 preferred_element_type=jnp.float32)


def _dense0_body(x_ref, pf_ref, px0_ref, px1_ref, wh0_ref, wh1_ref,
                 wf0_ref, wr_ref, bias_ref, out_ref, s_ref, ss_ref):
    i = pl.program_id(0)
    sf = pf_ref[0] + pf_ref[1]
    sx0 = px0_ref[0] + px0_ref[1]
    sx1 = px1_ref[0] + px1_ref[1]
    inv = 1.0 / jnp.maximum(sf[:, 0:1], 1.0)
    out = jnp.dot(sx0 * inv, wh0_ref[...], preferred_element_type=jnp.float32)
    out += jnp.dot(sx1 * inv, wh1_ref[...], preferred_element_type=jnp.float32)
    out += jnp.dot(sf * inv, wf0_ref[...], preferred_element_type=jnp.float32)
    out += jnp.dot(x_ref[...], wr_ref[...], preferred_element_type=jnp.float32)
    out += bias_ref[...]
    out_ref[...] = out

    @pl.when(i == 0)
    def _():
        s_ref[...] = jnp.zeros_like(s_ref)
        ss_ref[...] = jnp.zeros_like(ss_ref)

    s_ref[...] += jnp.sum(out, axis=0, keepdims=True)
    ss_ref[...] += jnp.sum(out * out, axis=0, keepdims=True)


def _bn_elu_p_body(out_ref, s_ref, ss_ref, g_ref, b_ref, w1h_ref,
                   h_ref, p_ref):
    m = s_ref[...] / N
    v = ss_ref[...] / N - m * m
    xn = (out_ref[...] - m) * lax.rsqrt(v + 1e-5) * g_ref[...] + b_ref[...]
    h = jnp.where(xn > 0, xn, jnp.exp(xn) - 1.0)
    h_ref[...] = h
    p_ref[...] = jnp.dot(h, w1h_ref[...], preferred_element_type=jnp.float32)


def _dense1_body(h_ref, pf_ref, pp_ref, wf1_ref, wr1_ref,
                 bias_ref, out_ref, s_ref, ss_ref):
    i = pl.program_id(0)
    sf = pf_ref[0] + pf_ref[1]
    sp = pp_ref[0] + pp_ref[1]
    inv = 1.0 / jnp.maximum(sf[:, 0:1], 1.0)
    out = sp * inv
    out += jnp.dot(sf * inv, wf1_ref[...], preferred_element_type=jnp.float32)
    out += jnp.dot(h_ref[...], wr1_ref[...], preferred_element_type=jnp.float32)
    out += bias_ref[...]
    out_ref[...] = out

    @pl.when(i == 0)
    def _():
        s_ref[...] = jnp.zeros_like(s_ref)
        ss_ref[...] = jnp.zeros_like(ss_ref)

    s_ref[...] += jnp.sum(out, axis=0, keepdims=True)
    ss_ref[...] += jnp.sum(out * out, axis=0, keepdims=True)


def _bn_elu_lsm_body(out_ref, s_ref, ss_ref, g_ref, b_ref, y_ref):
    m = s_ref[...] / N
    v = ss_ref[...] / N - m * m
    xn = (out_ref[...] - m) * lax.rsqrt(v + 1e-5) * g_ref[...] + b_ref[...]
    h = jnp.where(xn > 0, xn, jnp.exp(xn) - 1.0)
    mx = jnp.max(h, axis=1, keepdims=True)
    z = h - mx
    lse = jnp.log(jnp.sum(jnp.exp(z), axis=1, keepdims=True))
    y_ref[...] = z - lse


def kernel(x, edge_index, edge_attr, edge_t, edge_d, emb_type, emb_dir, t_w,
           t_b, l0_wl, l0_bl, l0_wr, l0_br, g0, b0, l1_wl, l1_bl, l1_wr,
           l1_br, g1, b1):
    f32 = jnp.float32

    src = edge_index[0].astype(jnp.int32)
    dst = edge_index[1].astype(jnp.int32)
    nw = NCORE * NSUB
    per_w = E // nw
    pad_w = CHUNKS * K - per_w
    pad_src = (jnp.arange(nw * pad_w, dtype=jnp.int32) * 83) % N
    pad_dst = N + jnp.arange(nw * pad_w, dtype=jnp.int32) % (N_PAD - N)

    def interleave(real, pad):
        return jnp.concatenate(
            [real.reshape(nw, per_w), pad.reshape(nw, pad_w)], axis=1
        ).reshape(NCORE, NSUB, CHUNKS, K)

    src4 = interleave(src, pad_src)
    dst4 = interleave(dst, pad_dst)
    shp2 = (NCORE, NSUB, CHUNKS2, K2)
    src4b = src4.reshape(shp2)
    dst4b = dst4.reshape(shp2)
    src_even = src4b * 2
    src_odd = src4b * 2 + 1
    zpad_f = jnp.zeros((nw * pad_w,), f32)
    zpad_i = jnp.zeros((nw * pad_w,), jnp.int32)
    t_pad = interleave(edge_t.astype(f32), zpad_f).reshape(GRID_E, 1, EB)
    a_pad = interleave(edge_attr.astype(jnp.int32), zpad_i) \
        .reshape(GRID_E, 1, EB)
    d_pad = interleave(edge_d.astype(jnp.int32), zpad_i) \
        .reshape(GRID_E, 1, EB)

    x2 = x.astype(f32).reshape(2 * N, 128)

    marr = jnp.arange(16, dtype=f32).reshape(16, 1)
    fact = jnp.cumprod(jnp.maximum(jnp.arange(16, dtype=f32), 1.0)) \
        .reshape(16, 1)
    twf = t_w.astype(f32).reshape(1, DT)
    tbf = t_b.astype(f32).reshape(1, DT)
    wpoly = (twf ** marr) * jnp.cos(tbf + marr * (jnp.pi / 2)) / fact
    wpoly = wpoly[:12]

    def wfeat(wl_ef, wl_et, dout):
        w = jnp.zeros((32, dout), f32)
        w = w.at[:12].set(wpoly @ wl_et.T.astype(f32))
        w = w.at[12:24].set(emb_type.astype(f32) @ wl_ef.T.astype(f32))
        w = w.at[24:26].set(emb_dir.astype(f32) @ wl_ef.T.astype(f32))
        return w

    wh0 = l0_wl[:, 0:128].T.astype(f32)
    wh1 = l0_wl[:, 128:256].T.astype(f32)
    wf0 = wfeat(l0_wl[:, DIN:DIN + DE], l0_wl[:, DIN + DE:DIN + DE + DT], DH)
    wr0 = l0_wr.T.astype(f32)
    bias0 = (l0_bl + l0_br).reshape(1, DH).astype(f32)
    w1h = l1_wl[:, :DH].T.astype(f32)
    wf1 = wfeat(l1_wl[:, DH:DH + DE], l1_wl[:, DH + DE:DH + DE + DT], DOUT)
    wr1 = l1_wr.T.astype(f32)
    bias1 = (l1_bl + l1_br).reshape(1, DOUT).astype(f32)
    g0r = g0.reshape(1, DH).astype(f32)
    b0r = b0.reshape(1, DH).astype(f32)
    g1r = g1.reshape(1, DOUT).astype(f32)
    b1r = b1.reshape(1, DOUT).astype(f32)

    zblk = jnp.zeros((ROWS_PER_SUB, 128), f32)
    z32 = jnp.zeros((ROWS_PER_SUB, 32), f32)
    eye32 = jnp.eye(32, dtype=f32)

    full = lambda shape: pl.BlockSpec(shape, lambda i: (0,) * len(shape))
    feat = pl.pallas_call(
        _feat_body,
        grid=(GRID_E,),
        in_specs=[
            pl.BlockSpec((1, 1, EB), lambda i: (i, 0, 0)),
            pl.BlockSpec((1, 1, EB), lambda i: (i, 0, 0)),
            pl.BlockSpec((1, 1, EB), lambda i: (i, 0, 0)),
            full((32, 32)),
        ],
        out_specs=pl.BlockSpec((EB, 32), lambda i: (i, 0)),
        out_shape=jax.ShapeDtypeStruct((E_PAD, 32), f32),
    )
    F = feat(t_pad, a_pad, d_pad, eye32)
    F5 = F.reshape(NCORE, NSUB, CHUNKS, K, 32)

    (pF,) = _make_scf()(F5, dst4, z32)
    pX0, pX1 = _make_scx()(x2, src_even, src_odd, dst4b, zblk)

    dense0 = pl.pallas_call(
        _dense0_body,
        grid=(GRID_N,),
        in_specs=[
            pl.BlockSpec((RB, DIN), lambda i: (i, 0)),
            pl.BlockSpec((NCORE, RB, 32), lambda i: (0, i, 0)),
            pl.BlockSpec((NCORE, RB, 128), lambda i: (0, i, 0)),
            pl.BlockSpec((NCORE, RB, 128), lambda i: (0, i, 0)),
            full((128, DH)), full((128, DH)), full((32, DH)),
            full((DIN, DH)), full((1, DH)),
        ],
        out_specs=[
            pl.BlockSpec((RB, DH), lambda i: (i, 0)),
            pl.BlockSpec((1, DH), lambda i: (0, 0)),
            pl.BlockSpec((1, DH), lambda i: (0, 0)),
        ],
        out_shape=[
            jax.ShapeDtypeStruct((N, DH), f32),
            jax.ShapeDtypeStruct((1, DH), f32),
            jax.ShapeDtypeStruct((1, DH), f32),
        ],
    )
    out0, s0, ss0 = dense0(x.astype(f32), pF, pX0, pX1, wh0, wh1,
                           wf0, wr0, bias0)

    bn0 = pl.pallas_call(
        _bn_elu_p_body,
        grid=(GRID_N,),
        in_specs=[
            pl.BlockSpec((RB, DH), lambda i: (i, 0)),
            full((1, DH)), full((1, DH)), full((1, DH)), full((1, DH)),
            full((DH, DOUT)),
        ],
        out_specs=[
            pl.BlockSpec((RB, DH), lambda i: (i, 0)),
            pl.BlockSpec((RB, DOUT), lambda i: (i, 0)),
        ],
        out_shape=[
            jax.ShapeDtypeStruct((N, DH), f32),
            jax.ShapeDtypeStruct((N, DOUT), f32),
        ],
    )
    h, p = bn0(out0, s0, ss0, g0r, b0r, w1h)

    sc2 = _make_sc2()
    (pP,) = sc2(p, src4b, dst4b, zblk)

    dense1 = pl.pallas_call(
        _dense1_body,
        grid=(GRID_N,),
        in_specs=[
            pl.BlockSpec((RB, DH), lambda i: (i, 0)),
            pl.BlockSpec((NCORE, RB, 32), lambda i: (0, i, 0)),
            pl.BlockSpec((NCORE, RB, 128), lambda i: (0, i, 0)),
            full((32, DOUT)), full((DH, DOUT)), full((1, DOUT)),
        ],
        out_specs=[
            pl.BlockSpec((RB, DOUT), lambda i: (i, 0)),
            pl.BlockSpec((1, DOUT), lambda i: (0, 0)),
            pl.BlockSpec((1, DOUT), lambda i: (0, 0)),
        ],
        out_shape=[
            jax.ShapeDtypeStruct((N, DOUT), f32),
            jax.ShapeDtypeStruct((1, DOUT), f32),
            jax.ShapeDtypeStruct((1, DOUT), f32),
        ],
    )
    out1, s1, ss1 = dense1(h, pF, pP, wf1, wr1, bias1)

    bn1 = pl.pallas_call(
        _bn_elu_lsm_body,
        grid=(GRID_N,),
        in_specs=[
            pl.BlockSpec((RB, DOUT), lambda i: (i, 0)),
            full((1, DOUT)), full((1, DOUT)), full((1, DOUT)), full((1, DOUT)),
        ],
        out_specs=pl.BlockSpec((RB, DOUT), lambda i: (i, 0)),
        out_shape=jax.ShapeDtypeStruct((N, DOUT), f32),
    )
    return bn1(out1, s1, ss1, g1r, b1r)
